# Initial kernel scaffold; baseline (speedup 1.0000x reference)
import jax
import jax.numpy as jnp
from jax.experimental import pallas as pl


def kernel(x, edge_index, W1, b1, W2, b2, W3, b3):
    raise NotImplementedError("write your pallas kernel here")


# speedup vs baseline: 11.2491x; 11.2491x over previous
import jax
import jax.numpy as jnp
from jax import lax
from jax.experimental import pallas as pl
from jax.experimental.pallas import tpu as pltpu
from jax.experimental.pallas import tpu_sc as plsc

NC = 2
NS = 16
NW = NC * NS
WIN = 80

_f32 = jnp.float32


def _sc_scatter_pass(n, d, e, gather):
    per_worker = e // NW
    nwin = per_worker // WIN
    assert per_worker * NW == e and nwin * WIN == per_worker, (n, d, e)
    nchunk = n // WIN
    assert nchunk * WIN == n
    chunk_iters = -(-nchunk // NS)

    mesh = plsc.VectorSubcoreMesh(core_axis_name="c", subcore_axis_name="s")

    def body(*refs):
        if gather:
            y_hbm, src_hbm, dst_hbm, out_hbm, src_v, dst_v, rows_v, acc_sh, sem = refs
        else:
            dst_hbm, out_hbm, dst_v, rows_v, acc_sh, sem = refs
        cid = lax.axis_index("c")
        sid = lax.axis_index("s")
        wid = sid * NC + cid

        zero16 = jnp.zeros((16,), _f32)

        @pl.loop(0, WIN)
        def _(i):
            for j0 in range(0, d, 16):
                j = min(j0, d - 16)
                rows_v[i, pl.ds(j, 16)] = zero16

        @pl.loop(0, chunk_iters)
        def _(t):
            c = sid + NS * t

            @pl.when(c < nchunk)
            def _():
                roff = pl.multiple_of(c * WIN, 8)
                pltpu.sync_copy(rows_v, acc_sh.at[pl.ds(roff, WIN)])

        if not gather:
            one16 = jnp.full((16,), 1.0, _f32)

            @pl.loop(0, WIN)
            def _(i):
                for j0 in range(0, d, 16):
                    j = min(j0, d - 16)
                    rows_v[i, pl.ds(j, 16)] = one16

        plsc.subcore_barrier()

        base = wid * per_worker

        @pl.loop(0, nwin)
        def _(w):
            off = pl.multiple_of(base + w * WIN, 8)
            pltpu.sync_copy(dst_hbm.at[pl.ds(off, WIN)], dst_v)
            if gather:
                pltpu.sync_copy(src_hbm.at[pl.ds(off, WIN)], src_v)
                pltpu.async_copy(y_hbm.at[src_v], rows_v, sem).wait()
            pltpu.sync_copy(rows_v, acc_sh.at[dst_v], add=True)

        plsc.subcore_barrier()

        @pl.loop(0, chunk_iters)
        def _(t):
            c = sid + NS * t

            @pl.when(c < nchunk)
            def _():
                roff = pl.multiple_of(c * WIN, 8)
                pltpu.sync_copy(acc_sh.at[pl.ds(roff, WIN)],
                                out_hbm.at[cid, pl.ds(roff, WIN)])

    scratch = []
    if gather:
        scratch.append(pltpu.VMEM((WIN,), jnp.int32))
    scratch += [
        pltpu.VMEM((WIN,), jnp.int32),
        pltpu.VMEM((WIN, d), _f32),
        pltpu.VMEM_SHARED((n, d), _f32),
        pltpu.SemaphoreType.DMA,
    ]
    return pl.kernel(
        body,
        out_type=jax.ShapeDtypeStruct((NC, n, d), _f32),
        mesh=mesh,
        scratch_types=scratch,
    )


def _tc_matmul_body(x_ref, w_ref, o_ref):
    o_ref[...] = jnp.dot(x_ref[...], w_ref[...],
                         preferred_element_type=_f32)


def _tc_scale_body(degp_ref, h_ref, y_ref, dinv_ref):
    deg = degp_ref[0, :, 0:1] + degp_ref[1, :, 0:1] + 1.0
    dinv = lax.rsqrt(deg)
    dinv_ref[...] = dinv
    y_ref[...] = h_ref[...] * dinv


def _tc_combine_body(p_ref, y_ref, dinv_ref, b_ref, w_ref, o_ref):
    agg = (p_ref[0] + p_ref[1] + y_ref[...]) * dinv_ref[...]
    h = jnp.maximum(agg + b_ref[...], 0.0)
    o_ref[...] = jnp.dot(h, w_ref[...], preferred_element_type=_f32) * dinv_ref[...]


def _tc_combine_nomm_body(p_ref, y_ref, dinv_ref, b_ref, o_ref):
    agg = (p_ref[0] + p_ref[1] + y_ref[...]) * dinv_ref[...]
    o_ref[...] = jnp.maximum(agg + b_ref[...], 0.0) * dinv_ref[...]


def _tc_final_body(p_ref, y_ref, dinv_ref, b_ref, w_ref, o_ref):
    agg = (p_ref[0] + p_ref[1] + y_ref[...]) * dinv_ref[...]
    z = jnp.dot(agg, w_ref[...], preferred_element_type=_f32) + b_ref[...]
    m = jnp.max(z, axis=-1, keepdims=True)
    t = z - m
    o_ref[...] = t - jnp.log(jnp.sum(jnp.exp(t), axis=-1, keepdims=True))


def kernel(x, edge_index, W1, b1, W2, b2, W3, b3):
    n, _ = x.shape
    e = edge_index.shape[1]
    nh = W1.shape[1]
    nc = W3.shape[1]
    src = edge_index[0].astype(jnp.int32)
    dst = edge_index[1].astype(jnp.int32)

    sds = jax.ShapeDtypeStruct

    h1 = pl.pallas_call(_tc_matmul_body, out_shape=sds((n, nh), _f32))(x, W1)
    degp = _sc_scatter_pass(n, 16, e, gather=False)(dst)

    y1, dinv = pl.pallas_call(
        _tc_scale_body,
        out_shape=(sds((n, nh), _f32), sds((n, 1), _f32)),
    )(degp, h1)

    p1 = _sc_scatter_pass(n, nh, e, gather=True)(y1, src, dst)
    y2 = pl.pallas_call(_tc_combine_body, out_shape=sds((n, nh), _f32))(
        p1, y1, dinv, b1.reshape(1, nh), W2)

    p2 = _sc_scatter_pass(n, nh, e, gather=True)(y2, src, dst)
    y3 = pl.pallas_call(_tc_combine_nomm_body, out_shape=sds((n, nh), _f32))(
        p2, y2, dinv, b2.reshape(1, nh))

    p3 = _sc_scatter_pass(n, nh, e, gather=True)(y3, src, dst)
    out = pl.pallas_call(_tc_final_body, out_shape=sds((n, nc), _f32))(
        p3, y3, dinv, b3.reshape(1, nc), W3)
    return out
